# baseline (speedup 1.0000x reference)
import functools

import jax
import jax.numpy as jnp
from jax import lax
from jax.experimental import pallas as pl
from jax.experimental.pallas import tpu as pltpu
from jax.experimental.pallas import tpu_sc as plsc

B = 4096
D = 128
A = 1000
G = 16
K = 16
NG = 1216
C = NG // K
P = NG * G

NC = 2
NS = 16
NW = NC * NS
BPW = B // NW


def _route(src_hbm, sidx_hbm, didx_hbm, out_hbm, src_v, dst_v, rows_v, sem):
    wid = lax.axis_index("s") * NC + lax.axis_index("c")
    base = wid * BPW
    pltpu.sync_copy(sidx_hbm.at[pl.ds(base, BPW)], src_v)
    pltpu.sync_copy(didx_hbm.at[pl.ds(base, BPW)], dst_v)
    pltpu.async_copy(src_hbm.at[src_v], rows_v, sem).wait()
    pltpu.async_copy(rows_v, out_hbm.at[dst_v], sem).wait()


def _make_route(n_out):
    mesh = plsc.VectorSubcoreMesh(core_axis_name="c", subcore_axis_name="s")
    return functools.partial(
        pl.kernel, mesh=mesh,
        out_type=jax.ShapeDtypeStruct((n_out, D), jnp.float32),
        scratch_types=[
            pltpu.VMEM((BPW,), jnp.int32),
            pltpu.VMEM((BPW,), jnp.int32),
            pltpu.VMEM((BPW, D), jnp.float32),
            pltpu.SemaphoreType.DMA,
        ],
    )(_route)


def _tc_body(garr_ref, x_ref, b_ref, w_hbm, o_ref, wbuf, sems):
    i = pl.program_id(0)

    def issue(step, phase):
        for k in range(K):
            widx = garr_ref[k * C + step]
            pltpu.make_async_copy(w_hbm.at[pl.ds(widx, 1)],
                                  wbuf.at[phase, pl.ds(k, 1)],
                                  sems.at[phase, k]).start()

    p = lax.rem(i, 2)

    @pl.when(i == 0)
    def _prime():
        issue(0, 0)

    @pl.when(i + 1 < C)
    def _next():
        issue(i + 1, 1 - p)

    for k in range(K):
        pltpu.make_async_copy(w_hbm.at[pl.ds(garr_ref[k * C + i], 1)],
                              wbuf.at[p, pl.ds(k, 1)],
                              sems.at[p, k]).wait()
        bidx = garr_ref[k * C + i]
        xg = x_ref[k, 0]
        w = wbuf[p, k]
        z = jax.lax.dot_general(xg, w, (((1,), (1,)), ((), ())),
                                preferred_element_type=jnp.float32)
        o_ref[k, 0] = z + b_ref[pl.ds(bidx, 1), :]


def kernel(x, a, weight, bias):
    idx = a[:, 0].astype(jnp.int32)
    iota = jnp.arange(B, dtype=jnp.int32)
    sidx, perm = jax.lax.sort_key_val(idx, iota)

    starts = jnp.concatenate([jnp.ones((1,), jnp.bool_),
                              sidx[1:] != sidx[:-1]])
    seg_start = jax.lax.cummax(jnp.where(starts, iota, 0))
    pos_in_run = iota - seg_start
    new_group = starts | (pos_in_run % G == 0)
    g = jnp.cumsum(new_group.astype(jnp.int32)) - 1
    ppos = g * G + pos_in_run % G

    key = jnp.where(new_group, g, jnp.int32(1 << 20))
    val = jnp.where(new_group, sidx, jnp.int32(0))
    _, garr = jax.lax.sort_key_val(key, val)
    garr = garr[:NG]

    x_pad = _make_route(P)(x, perm, ppos)

    z_pad = pl.pallas_call(
        _tc_body,
        grid_spec=pltpu.PrefetchScalarGridSpec(
            num_scalar_prefetch=1,
            grid=(C,),
            in_specs=[
                pl.BlockSpec((K, 1, G, D), lambda i, g_: (0, i, 0, 0)),
                pl.BlockSpec((A, D), lambda i, g_: (0, 0)),
                pl.BlockSpec(memory_space=pltpu.HBM),
            ],
            out_specs=pl.BlockSpec((K, 1, G, D), lambda i, g_: (0, i, 0, 0)),
            scratch_shapes=[
                pltpu.VMEM((2, K, D, D), jnp.float32),
                pltpu.SemaphoreType.DMA((2, K)),
            ],
        ),
        out_shape=jax.ShapeDtypeStruct((K, C, G, D), jnp.float32),
    )(garr, x_pad.reshape(K, C, G, D), bias, weight)

    return _make_route(B)(z_pad.reshape(P, D), ppos, perm)


# speedup vs baseline: 2.5283x; 2.5283x over previous
import functools

import jax
import jax.numpy as jnp
from jax import lax
from jax.experimental import pallas as pl
from jax.experimental.pallas import tpu as pltpu
from jax.experimental.pallas import tpu_sc as plsc

B = 4096
D = 128
A = 1000
G = 16
K = 32
NG = 1216
C = NG // K
P = NG * G

NC = 2
NS = 16
NW = NC * NS
BPW = B // NW


def _route(src_hbm, sidx_hbm, didx_hbm, out_hbm, src_v, dst_v, rows_v, sem):
    wid = lax.axis_index("s") * NC + lax.axis_index("c")
    base = wid * BPW
    pltpu.sync_copy(sidx_hbm.at[pl.ds(base, BPW)], src_v)
    pltpu.sync_copy(didx_hbm.at[pl.ds(base, BPW)], dst_v)
    pltpu.async_copy(src_hbm.at[src_v], rows_v, sem).wait()
    pltpu.async_copy(rows_v, out_hbm.at[dst_v], sem).wait()


def _make_route(n_out):
    mesh = plsc.VectorSubcoreMesh(core_axis_name="c", subcore_axis_name="s")
    return functools.partial(
        pl.kernel, mesh=mesh,
        out_type=jax.ShapeDtypeStruct((n_out, D), jnp.float32),
        scratch_types=[
            pltpu.VMEM((BPW,), jnp.int32),
            pltpu.VMEM((BPW,), jnp.int32),
            pltpu.VMEM((BPW, D), jnp.float32),
            pltpu.SemaphoreType.DMA,
        ],
    )(_route)


def _tc_body(garr_ref, x_ref, b_ref, *rest):
    w_refs = rest[:K]
    o_ref = rest[K]
    i = pl.program_id(0)
    for k in range(K):
        bidx = garr_ref[k * C + i]
        xg = x_ref[k, 0].astype(jnp.bfloat16)
        w = w_refs[k][0].astype(jnp.bfloat16)
        z = jax.lax.dot_general(xg, w, (((1,), (1,)), ((), ())),
                                preferred_element_type=jnp.float32)
        o_ref[k, 0] = z + b_ref[pl.ds(bidx, 1), :]


def kernel(x, a, weight, bias):
    idx = a[:, 0].astype(jnp.int32)
    iota = jnp.arange(B, dtype=jnp.int32)
    sidx, perm = jax.lax.sort_key_val(idx, iota)

    starts = jnp.concatenate([jnp.ones((1,), jnp.bool_),
                              sidx[1:] != sidx[:-1]])
    seg_start = jax.lax.cummax(jnp.where(starts, iota, 0))
    pos_in_run = iota - seg_start
    new_group = starts | (pos_in_run % G == 0)
    g = jnp.cumsum(new_group.astype(jnp.int32)) - 1
    ppos = g * G + pos_in_run % G

    key = jnp.where(new_group, g, jnp.int32(1 << 20))
    val = jnp.where(new_group, sidx, jnp.int32(0))
    _, garr = jax.lax.sort_key_val(key, val)
    garr = garr[:NG]

    x_pad = _make_route(P)(x, perm, ppos)

    def w_spec(k):
        return pl.BlockSpec(
            (1, D, D),
            lambda i, g_ref, _k=k: (g_ref[_k * C + i], 0, 0))

    z_pad = pl.pallas_call(
        _tc_body,
        grid_spec=pltpu.PrefetchScalarGridSpec(
            num_scalar_prefetch=1,
            grid=(C,),
            in_specs=[
                pl.BlockSpec((K, 1, G, D), lambda i, g_: (0, i, 0, 0)),
                pl.BlockSpec((A, D), lambda i, g_: (0, 0)),
            ] + [w_spec(k) for k in range(K)],
            out_specs=pl.BlockSpec((K, 1, G, D), lambda i, g_: (0, i, 0, 0)),
        ),
        out_shape=jax.ShapeDtypeStruct((K, C, G, D), jnp.float32),
    )(garr, x_pad.reshape(K, C, G, D), bias, *([weight] * K))

    return _make_route(B)(z_pad.reshape(P, D), ppos, perm)


# speedup vs baseline: 2.5394x; 1.0044x over previous
import functools

import jax
import jax.numpy as jnp
from jax import lax
from jax.experimental import pallas as pl
from jax.experimental.pallas import tpu as pltpu
from jax.experimental.pallas import tpu_sc as plsc

B = 4096
D = 128
A = 1000
G = 16
K = 64
NG = 1280
C = NG // K
P = NG * G

NC = 2
NS = 16
NW = NC * NS
BPW = B // NW


def _route(src_hbm, sidx_hbm, didx_hbm, out_hbm, src_v, dst_v, rows_v, sem):
    wid = lax.axis_index("s") * NC + lax.axis_index("c")
    base = wid * BPW
    pltpu.sync_copy(sidx_hbm.at[pl.ds(base, BPW)], src_v)
    pltpu.sync_copy(didx_hbm.at[pl.ds(base, BPW)], dst_v)
    pltpu.async_copy(src_hbm.at[src_v], rows_v, sem).wait()
    pltpu.async_copy(rows_v, out_hbm.at[dst_v], sem).wait()


def _make_route(n_out):
    mesh = plsc.VectorSubcoreMesh(core_axis_name="c", subcore_axis_name="s")
    return functools.partial(
        pl.kernel, mesh=mesh,
        out_type=jax.ShapeDtypeStruct((n_out, D), jnp.float32),
        scratch_types=[
            pltpu.VMEM((BPW,), jnp.int32),
            pltpu.VMEM((BPW,), jnp.int32),
            pltpu.VMEM((BPW, D), jnp.float32),
            pltpu.SemaphoreType.DMA,
        ],
    )(_route)


def _tc_body(garr_ref, x_ref, b_ref, *rest):
    w_refs = rest[:K]
    o_ref = rest[K]
    i = pl.program_id(0)
    for k in range(K):
        bidx = garr_ref[k * C + i]
        xg = x_ref[k, 0]
        z = jax.lax.dot_general(xg, w_refs[k][0], (((1,), (1,)), ((), ())),
                                preferred_element_type=jnp.float32)
        o_ref[k, 0] = z + b_ref[pl.ds(bidx, 1), :]


def kernel(x, a, weight, bias):
    idx = a[:, 0].astype(jnp.int32)
    iota = jnp.arange(B, dtype=jnp.int32)
    sidx, perm = jax.lax.sort_key_val(idx, iota)

    starts = jnp.concatenate([jnp.ones((1,), jnp.bool_),
                              sidx[1:] != sidx[:-1]])
    seg_start = jax.lax.cummax(jnp.where(starts, iota, 0))
    pos_in_run = iota - seg_start
    new_group = starts | (pos_in_run % G == 0)
    g = jnp.cumsum(new_group.astype(jnp.int32)) - 1
    ppos = g * G + pos_in_run % G

    key = jnp.where(new_group, g, jnp.int32(1 << 20))
    val = jnp.where(new_group, sidx, jnp.int32(0))
    _, garr = jax.lax.sort_key_val(key, val)
    garr = garr[:NG]

    x_pad = _make_route(P)(x, perm, ppos)

    def w_spec(k):
        return pl.BlockSpec(
            (1, D, D),
            lambda i, g_ref, _k=k: (g_ref[_k * C + i], 0, 0))

    z_pad = pl.pallas_call(
        _tc_body,
        grid_spec=pltpu.PrefetchScalarGridSpec(
            num_scalar_prefetch=1,
            grid=(C,),
            in_specs=[
                pl.BlockSpec((K, 1, G, D), lambda i, g_: (0, i, 0, 0)),
                pl.BlockSpec((A, D), lambda i, g_: (0, 0)),
            ] + [w_spec(k) for k in range(K)],
            out_specs=pl.BlockSpec((K, 1, G, D), lambda i, g_: (0, i, 0, 0)),
        ),
        out_shape=jax.ShapeDtypeStruct((K, C, G, D), jnp.float32),
    )(garr, x_pad.reshape(K, C, G, D), bias, *([weight] * K))

    return _make_route(B)(z_pad.reshape(P, D), ppos, perm)


# speedup vs baseline: 2.8112x; 1.1070x over previous
import functools

import jax
import jax.numpy as jnp
from jax import lax
from jax.experimental import pallas as pl
from jax.experimental.pallas import tpu as pltpu
from jax.experimental.pallas import tpu_sc as plsc

B = 4096
D = 128
A = 1000
G = 16
K = 64
NG = 1280
C = NG // K
P = NG * G

NC = 2
NS = 16
NW = NC * NS
BPW = B // NW


def _route(src_hbm, sidx_hbm, didx_hbm, out_hbm, src_v, dst_v, rows_v, sem):
    wid = lax.axis_index("s") * NC + lax.axis_index("c")
    base = wid * BPW
    pltpu.sync_copy(sidx_hbm.at[pl.ds(base, BPW)], src_v)
    pltpu.sync_copy(didx_hbm.at[pl.ds(base, BPW)], dst_v)
    pltpu.async_copy(src_hbm.at[src_v], rows_v, sem).wait()
    pltpu.async_copy(rows_v, out_hbm.at[dst_v], sem).wait()


def _make_route(n_out):
    mesh = plsc.VectorSubcoreMesh(core_axis_name="c", subcore_axis_name="s")
    return functools.partial(
        pl.kernel, mesh=mesh,
        out_type=jax.ShapeDtypeStruct((n_out, D), jnp.float32),
        scratch_types=[
            pltpu.VMEM((BPW,), jnp.int32),
            pltpu.VMEM((BPW,), jnp.int32),
            pltpu.VMEM((BPW, D), jnp.float32),
            pltpu.SemaphoreType.DMA,
        ],
    )(_route)


def _tc_body(garr_ref, x_ref, b_ref, *rest):
    w_refs = rest[:K]
    o_ref = rest[K]
    i = pl.program_id(0)
    for k in range(K):
        bidx = garr_ref[k * C + i]
        xg = x_ref[k, 0]
        z = jax.lax.dot_general(xg, w_refs[k][0], (((1,), (1,)), ((), ())),
                                preferred_element_type=jnp.float32)
        o_ref[k, 0] = z + b_ref[pl.ds(bidx, 1), :]


def kernel(x, a, weight, bias):
    idx = a[:, 0].astype(jnp.int32)
    iota = jnp.arange(B, dtype=jnp.int32)
    sidx, perm = jax.lax.sort_key_val(idx, iota)

    starts = jnp.concatenate([jnp.ones((1,), jnp.bool_),
                              sidx[1:] != sidx[:-1]])
    seg_start = jax.lax.cummax(jnp.where(starts, iota, 0))
    pos_in_run = iota - seg_start
    new_group = starts | (pos_in_run % G == 0)
    g = jnp.cumsum(new_group.astype(jnp.int32)) - 1
    ppos = g * G + pos_in_run % G

    key = jnp.where(new_group, g, jnp.int32(1 << 20))
    val = jnp.where(new_group, sidx, jnp.int32(0))
    _, garr = jax.lax.sort_key_val(key, val)
    garr = garr[:NG]
    n_groups = g[-1] + 1
    c_dyn = jnp.minimum((n_groups + K - 1) // K, C)

    x_pad = _make_route(P)(x, perm, ppos)

    def w_spec(k):
        return pl.BlockSpec(
            (1, D, D),
            lambda i, g_ref, _k=k: (g_ref[_k * C + i], 0, 0))

    z_pad = pl.pallas_call(
        _tc_body,
        grid_spec=pltpu.PrefetchScalarGridSpec(
            num_scalar_prefetch=1,
            grid=(c_dyn,),
            in_specs=[
                pl.BlockSpec((K, 1, G, D), lambda i, g_: (0, i, 0, 0)),
                pl.BlockSpec((A, D), lambda i, g_: (0, 0)),
            ] + [w_spec(k) for k in range(K)],
            out_specs=pl.BlockSpec((K, 1, G, D), lambda i, g_: (0, i, 0, 0)),
        ),
        out_shape=jax.ShapeDtypeStruct((K, C, G, D), jnp.float32),
    )(garr, x_pad.reshape(K, C, G, D), bias, *([weight] * K))

    return _make_route(B)(z_pad.reshape(P, D), ppos, perm)
